# baseline (speedup 1.0000x reference)
import jax
import jax.numpy as jnp
from jax.experimental import pallas as pl
from jax.experimental.pallas import tpu as pltpu

N_RES = 38
BN = 5000


def _body(x_ref, w_ref, b_ref, o_ref):
    xb = x_ref[...]
    cols = jax.lax.broadcasted_iota(jnp.int32, xb.shape, 1)
    in_head = cols < N_RES
    head = jnp.where(in_head, xb, -jnp.inf)
    mx = jnp.max(head, axis=1, keepdims=True)
    idx = jnp.min(jnp.where(head == mx, cols, jnp.int32(10**9)),
                  axis=1, keepdims=True)
    onehot_or_x = jnp.where(in_head, (cols == idx).astype(xb.dtype), xb)
    o_ref[...] = (
        jnp.dot(onehot_or_x, w_ref[...], preferred_element_type=jnp.float32)
        + b_ref[...]
    )


def kernel(x, emb_table, W_feats, b_feats, W_sigma, b_sigma):
    n, d = x.shape
    n_s = emb_table.shape[1]
    dp = 384
    xp = jnp.pad(x, ((0, 0), (0, dp - d)))
    w_full = jnp.concatenate(
        [emb_table, W_feats.T, W_sigma.T,
         jnp.zeros((dp - d, n_s), jnp.float32)], axis=0)
    bias = (b_feats + b_sigma)[None, :]
    return pl.pallas_call(
        _body,
        grid=(n // BN,),
        in_specs=[
            pl.BlockSpec((BN, dp), lambda i: (i, 0)),
            pl.BlockSpec((dp, n_s), lambda i: (0, 0)),
            pl.BlockSpec((1, n_s), lambda i: (0, 0)),
        ],
        out_specs=pl.BlockSpec((BN, n_s), lambda i: (i, 0)),
        out_shape=jax.ShapeDtypeStruct((n, n_s), jnp.float32),
        compiler_params=pltpu.CompilerParams(
            dimension_semantics=("parallel",),
        ),
    )(xp, w_full, bias)


# speedup vs baseline: 2.2214x; 2.2214x over previous
import functools
import jax
import jax.numpy as jnp
from jax import lax
from jax.experimental import pallas as pl
from jax.experimental.pallas import tpu as pltpu
from jax.experimental.pallas import tpu_sc as plsc

N_RES = 38
CR = 160
NW = 32
BN = 4000


def _sc_argmax(x):
    n, d = x.shape
    nchunk = n // CR
    mesh = plsc.VectorSubcoreMesh(core_axis_name="c", subcore_axis_name="s")

    @functools.partial(
        pl.kernel,
        mesh=mesh,
        out_type=jax.ShapeDtypeStruct((n,), jnp.int32),
        scratch_types=[
            pltpu.VMEM((2, CR, 128), jnp.float32),
            pltpu.VMEM((CR,), jnp.int32),
            pltpu.SemaphoreType.DMA((2,)),
        ],
    )
    def k(x_hbm, out_hbm, bufs, idxbuf, sems):
        cid = lax.axis_index("c")
        sid = lax.axis_index("s")
        wid = sid * 2 + cid
        trips = (nchunk - 1 - wid) // NW + 1
        lanes = lax.iota(jnp.int32, 16)
        neg = jnp.full((16,), -jnp.inf, jnp.float32)
        tail_mask = lanes < (N_RES - 32)
        lane_b = lanes + 16
        lane_c = lanes + 32
        big = jnp.full((16,), 10 ** 9, jnp.int32)

        rots = [(lanes + k) % 16 for k in (1, 2, 4, 8)]

        def allmax(v):
            for rot in rots:
                v = jnp.maximum(v, jnp.take(v, rot))
            return v

        def start(t, slot):
            chunk = wid + t * NW
            pltpu.make_async_copy(
                x_hbm.at[pl.ds(chunk * CR, CR), pl.ds(0, 128)],
                bufs.at[slot], sems.at[slot],
            ).start()

        start(0, 0)

        def body(t, carry):
            slot = lax.rem(t, 2)

            @pl.when(t + 1 < trips)
            def _pref():
                start(t + 1, 1 - slot)

            chunk = wid + t * NW
            pltpu.make_async_copy(
                x_hbm.at[pl.ds(chunk * CR, CR), pl.ds(0, 128)],
                bufs.at[slot], sems.at[slot],
            ).wait()
            buf = bufs.at[slot]
            for r0 in range(0, CR, 16):
                acc = big
                for j in range(16):
                    r = r0 + j
                    va = buf[r, pl.ds(0, 16)]
                    vb = buf[r, pl.ds(16, 16)]
                    vc = jnp.where(tail_mask, buf[r, pl.ds(32, 16)], neg)
                    m1 = jnp.maximum(va, vb)
                    i1 = jnp.where(vb > va, lane_b, lanes)
                    m2 = jnp.maximum(m1, vc)
                    i2 = jnp.where(vc > m1, lane_c, i1)
                    best = allmax(m2)
                    cand = jnp.where(m2 == best, i2, big)
                    col = -allmax(-cand)
                    acc = jnp.where(lanes == j, col, acc)
                idxbuf[pl.ds(r0, 16)] = acc
            pltpu.sync_copy(idxbuf, out_hbm.at[pl.ds(chunk * CR, CR)])
            return carry

        lax.fori_loop(0, trips, body, 0)

    return k(x)


def _tc_body(x_ref, idx_ref, w_ref, b_ref, o_ref):
    xb = x_ref[...]
    idxb = idx_ref[...]
    cols = jax.lax.broadcasted_iota(jnp.int32, xb.shape, 1)
    in_head = cols < N_RES
    onehot_or_x = jnp.where(in_head, (cols == idxb).astype(xb.dtype), xb)
    o_ref[...] = (
        jnp.dot(onehot_or_x, w_ref[...], preferred_element_type=jnp.float32)
        + b_ref[...]
    )


def kernel(x, emb_table, W_feats, b_feats, W_sigma, b_sigma):
    n, d = x.shape
    n_s = emb_table.shape[1]
    idx = _sc_argmax(x).reshape(n, 1)
    w_full = jnp.concatenate([emb_table, W_feats.T, W_sigma.T], axis=0)
    bias = (b_feats + b_sigma)[None, :]
    return pl.pallas_call(
        _tc_body,
        grid=(n // BN,),
        in_specs=[
            pl.BlockSpec((BN, d), lambda i: (i, 0)),
            pl.BlockSpec((BN, 1), lambda i: (i, 0)),
            pl.BlockSpec((d, n_s), lambda i: (0, 0)),
            pl.BlockSpec((1, n_s), lambda i: (0, 0)),
        ],
        out_specs=pl.BlockSpec((BN, n_s), lambda i: (i, 0)),
        out_shape=jax.ShapeDtypeStruct((n, n_s), jnp.float32),
        compiler_params=pltpu.CompilerParams(
            dimension_semantics=("parallel",),
        ),
    )(x, idx, w_full, bias)
